# Initial kernel scaffold; baseline (speedup 1.0000x reference)
import jax
import jax.numpy as jnp
from jax.experimental import pallas as pl


def kernel(x, batch, W1, b1, W2, b2):
    raise NotImplementedError("write your pallas kernel here")


# speedup vs baseline: 11.8659x; 11.8659x over previous
import functools

import jax
import jax.numpy as jnp
from jax.experimental import pallas as pl
from jax.experimental.pallas import tpu as pltpu

_BLOCK = 2000


def _pool_kernel(x_ref, b_ref, w1_ref, b1_ref, w2_ref, b2_ref, out_ref,
                 m_ref, d_ref, acc_ref, *, num_segments):
    i = pl.program_id(0)
    nb = pl.num_programs(0)
    g = num_segments

    @pl.when(i == 0)
    def _init():
        m_ref[...] = jnp.full(m_ref.shape, -jnp.inf, dtype=jnp.float32)
        d_ref[...] = jnp.zeros(d_ref.shape, dtype=jnp.float32)
        acc_ref[...] = jnp.zeros(acc_ref.shape, dtype=jnp.float32)

    x = x_ref[...]
    seg = b_ref[0]
    bsz = x.shape[0]

    h = jnp.tanh(
        jnp.dot(x, w1_ref[...], preferred_element_type=jnp.float32)
        + b1_ref[...])
    s_t = jax.lax.dot_general(
        w2_ref[...], h, (((1,), (1,)), ((), ())),
        preferred_element_type=jnp.float32) + b2_ref[...]

    seg_ids = jax.lax.broadcasted_iota(jnp.int32, (g, bsz), 0)
    mask = seg_ids == seg

    neg_inf = jnp.float32(-jnp.inf)
    bm = jnp.max(jnp.where(mask, s_t, neg_inf), axis=1, keepdims=True)
    m_old = m_ref[...]
    m_new = jnp.maximum(m_old, bm)
    m_safe = jnp.where(m_new == neg_inf, 0.0, m_new)
    scale = jnp.exp(m_old - m_safe)

    m_node = jnp.max(jnp.where(mask, m_safe, neg_inf), axis=0,
                     keepdims=True)
    e_t = jnp.exp(s_t - m_node)
    p = jnp.where(mask, e_t, 0.0)

    m_ref[...] = m_new
    d_ref[...] = d_ref[...] * scale + jnp.sum(p, axis=1, keepdims=True)
    acc_ref[...] = acc_ref[...] * scale + jnp.dot(
        p, x, preferred_element_type=jnp.float32)

    @pl.when(i == nb - 1)
    def _finish():
        d = d_ref[...]
        out_ref[...] = acc_ref[...] / jnp.where(d > 0, d, 1.0)


def kernel(x, batch, W1, b1, W2, b2):
    n, d = x.shape
    num_segments = 64
    block = _BLOCK
    nb = n // block
    assert nb * block == n

    batch32 = batch.astype(jnp.int32).reshape(nb, 1, block)
    b1r = b1.reshape(1, d)
    w2r = W2.reshape(1, d)
    b2r = b2.reshape(1, 1)

    grid_spec = pltpu.PrefetchScalarGridSpec(
        num_scalar_prefetch=0,
        grid=(nb,),
        in_specs=[
            pl.BlockSpec((block, d), lambda i: (i, 0)),
            pl.BlockSpec((1, 1, block), lambda i: (i, 0, 0)),
            pl.BlockSpec((d, d), lambda i: (0, 0)),
            pl.BlockSpec((1, d), lambda i: (0, 0)),
            pl.BlockSpec((1, d), lambda i: (0, 0)),
            pl.BlockSpec((1, 1), lambda i: (0, 0)),
        ],
        out_specs=pl.BlockSpec((num_segments, d), lambda i: (0, 0)),
        scratch_shapes=[
            pltpu.VMEM((num_segments, 1), jnp.float32),
            pltpu.VMEM((num_segments, 1), jnp.float32),
            pltpu.VMEM((num_segments, d), jnp.float32),
        ],
    )

    return pl.pallas_call(
        functools.partial(_pool_kernel, num_segments=num_segments),
        grid_spec=grid_spec,
        out_shape=jax.ShapeDtypeStruct((num_segments, d), jnp.float32),
        compiler_params=pltpu.CompilerParams(
            dimension_semantics=("arbitrary",),
        ),
    )(x, batch32, W1, b1r, w2r, b2r)


# speedup vs baseline: 16.5719x; 1.3966x over previous
import functools

import jax
import jax.numpy as jnp
from jax.experimental import pallas as pl
from jax.experimental.pallas import tpu as pltpu

_BLOCK = 4000


def _pool_kernel(x_ref, b_ref, w1_ref, b1_ref, w2_ref, b2_ref, out_ref,
                 m_ref, d_ref, acc_ref, *, num_segments):
    i = pl.program_id(0)
    nb = pl.num_programs(0)
    g = num_segments

    @pl.when(i == 0)
    def _init():
        m_ref[...] = jnp.full(m_ref.shape, -jnp.inf, dtype=jnp.float32)
        d_ref[...] = jnp.zeros(d_ref.shape, dtype=jnp.float32)
        acc_ref[...] = jnp.zeros(acc_ref.shape, dtype=jnp.float32)

    x = x_ref[...]
    xb = x.astype(jnp.bfloat16)
    seg = b_ref[0]
    bsz = x.shape[0]

    h = jnp.tanh(
        jnp.dot(xb, w1_ref[...], preferred_element_type=jnp.float32)
        + b1_ref[...])
    s_t = jax.lax.dot_general(
        w2_ref[...], h, (((1,), (1,)), ((), ())),
        preferred_element_type=jnp.float32) + b2_ref[...]

    seg_ids = jax.lax.broadcasted_iota(jnp.int32, (g, bsz), 0)
    mask = seg_ids == seg
    mask_f = mask.astype(jnp.float32)

    neg_inf = jnp.float32(-jnp.inf)
    bm = jnp.max(jnp.where(mask, s_t, neg_inf), axis=1, keepdims=True)
    m_old = m_ref[...]
    m_new = jnp.maximum(m_old, bm)
    m_safe = jnp.where(m_new == neg_inf, 0.0, m_new)
    scale = jnp.exp(m_old - m_safe)

    m_node = jax.lax.dot_general(
        m_safe, mask_f, (((0,), (0,)), ((), ())),
        preferred_element_type=jnp.float32)
    e_t = jnp.exp(s_t - m_node)
    p = mask_f * e_t

    m_ref[...] = m_new
    d_ref[...] = d_ref[...] * scale + jnp.sum(p, axis=1, keepdims=True)
    acc_ref[...] = acc_ref[...] * scale + jnp.dot(
        p.astype(jnp.bfloat16), xb,
        preferred_element_type=jnp.float32)

    @pl.when(i == nb - 1)
    def _finish():
        d = d_ref[...]
        out_ref[...] = acc_ref[...] / jnp.where(d > 0, d, 1.0)


def kernel(x, batch, W1, b1, W2, b2):
    n, d = x.shape
    num_segments = 64
    block = _BLOCK
    nb = n // block
    assert nb * block == n

    batch32 = batch.astype(jnp.int32).reshape(nb, 1, block)
    w1b = W1.astype(jnp.bfloat16)
    b1r = b1.reshape(1, d)
    w2r = W2.reshape(1, d)
    b2r = b2.reshape(1, 1)

    grid_spec = pltpu.PrefetchScalarGridSpec(
        num_scalar_prefetch=0,
        grid=(nb,),
        in_specs=[
            pl.BlockSpec((block, d), lambda i: (i, 0)),
            pl.BlockSpec((1, 1, block), lambda i: (i, 0, 0)),
            pl.BlockSpec((d, d), lambda i: (0, 0)),
            pl.BlockSpec((1, d), lambda i: (0, 0)),
            pl.BlockSpec((1, d), lambda i: (0, 0)),
            pl.BlockSpec((1, 1), lambda i: (0, 0)),
        ],
        out_specs=pl.BlockSpec((num_segments, d), lambda i: (0, 0)),
        scratch_shapes=[
            pltpu.VMEM((num_segments, 1), jnp.float32),
            pltpu.VMEM((num_segments, 1), jnp.float32),
            pltpu.VMEM((num_segments, d), jnp.float32),
        ],
    )

    return pl.pallas_call(
        functools.partial(_pool_kernel, num_segments=num_segments),
        grid_spec=grid_spec,
        out_shape=jax.ShapeDtypeStruct((num_segments, d), jnp.float32),
        compiler_params=pltpu.CompilerParams(
            dimension_semantics=("arbitrary",),
        ),
    )(x, batch32, w1b, b1r, w2r, b2r)


# speedup vs baseline: 16.7663x; 1.0117x over previous
import functools

import jax
import jax.numpy as jnp
from jax.experimental import pallas as pl
from jax.experimental.pallas import tpu as pltpu

_BLOCK = 4000


def _pool_kernel(x_ref, b_ref, w1_ref, b1_ref, w2_ref, b2_ref, out_ref,
                 m_ref, d_ref, acc_ref, *, num_segments):
    i = pl.program_id(0)
    nb = pl.num_programs(0)
    g = num_segments

    @pl.when(i == 0)
    def _init():
        m_ref[...] = jnp.full(m_ref.shape, -jnp.inf, dtype=jnp.float32)
        d_ref[...] = jnp.zeros(d_ref.shape, dtype=jnp.float32)
        acc_ref[...] = jnp.zeros(acc_ref.shape, dtype=jnp.float32)

    x = x_ref[...]
    xb = x.astype(jnp.bfloat16)
    seg = b_ref[0]
    bsz = x.shape[0]

    h = jnp.tanh(
        jnp.dot(xb, w1_ref[...], preferred_element_type=jnp.float32)
        + b1_ref[...])
    s_t = jax.lax.dot_general(
        w2_ref[...], h, (((1,), (1,)), ((), ())),
        preferred_element_type=jnp.float32) + b2_ref[...]

    blk_max = jnp.max(s_t)
    m_old = m_ref[...]
    m_new = jnp.maximum(m_old, blk_max)
    scale_old = jnp.exp(m_old - m_new)
    scale_blk = jnp.exp(blk_max - m_new)

    e_t = jnp.exp(s_t - blk_max)
    seg_ids = jax.lax.broadcasted_iota(jnp.int32, (g, bsz), 0)
    p = jnp.where(seg_ids == seg, e_t, 0.0).astype(jnp.bfloat16)

    pd = jnp.dot(p, jnp.ones((bsz, 1), jnp.bfloat16),
                 preferred_element_type=jnp.float32)
    pa = jnp.dot(p, xb, preferred_element_type=jnp.float32)

    m_ref[...] = m_new
    d_ref[...] = d_ref[...] * scale_old + scale_blk * pd
    acc_ref[...] = acc_ref[...] * scale_old + scale_blk * pa

    @pl.when(i == nb - 1)
    def _finish():
        d = d_ref[...]
        out_ref[...] = acc_ref[...] / jnp.where(d > 0, d, 1.0)


def kernel(x, batch, W1, b1, W2, b2):
    n, d = x.shape
    num_segments = 64
    block = _BLOCK
    nb = n // block
    assert nb * block == n

    batch32 = batch.astype(jnp.int32).reshape(nb, 1, block)
    w1b = W1.astype(jnp.bfloat16)
    b1r = b1.reshape(1, d)
    w2r = W2.reshape(1, d)
    b2r = b2.reshape(1, 1)

    grid_spec = pltpu.PrefetchScalarGridSpec(
        num_scalar_prefetch=0,
        grid=(nb,),
        in_specs=[
            pl.BlockSpec((block, d), lambda i: (i, 0)),
            pl.BlockSpec((1, 1, block), lambda i: (i, 0, 0)),
            pl.BlockSpec((d, d), lambda i: (0, 0)),
            pl.BlockSpec((1, d), lambda i: (0, 0)),
            pl.BlockSpec((1, d), lambda i: (0, 0)),
            pl.BlockSpec((1, 1), lambda i: (0, 0)),
        ],
        out_specs=pl.BlockSpec((num_segments, d), lambda i: (0, 0)),
        scratch_shapes=[
            pltpu.VMEM((num_segments, 1), jnp.float32),
            pltpu.VMEM((num_segments, 1), jnp.float32),
            pltpu.VMEM((num_segments, d), jnp.float32),
        ],
    )

    return pl.pallas_call(
        functools.partial(_pool_kernel, num_segments=num_segments),
        grid_spec=grid_spec,
        out_shape=jax.ShapeDtypeStruct((num_segments, d), jnp.float32),
        compiler_params=pltpu.CompilerParams(
            dimension_semantics=("arbitrary",),
        ),
    )(x, batch32, w1b, b1r, w2r, b2r)
